# baseline (speedup 1.0000x reference)
import functools

import jax
import jax.numpy as jnp
from jax import lax
from jax.experimental import pallas as pl
from jax.experimental.pallas import tpu as pltpu
from jax.experimental.pallas import tpu_sc as plsc

_NC, _NS, _L = 2, 16, 16
_NW = _NC * _NS


def _sc_add_kernel(x1, mask16, w):
    N = x1.shape[0]
    D = 2 * w.shape[1]
    nvec = D // _L
    per_w = N // _NW
    RCH = 8
    CH = RCH * D
    nch = per_w // CH
    NBUF = 4
    mesh = plsc.VectorSubcoreMesh(core_axis_name="c", subcore_axis_name="s")

    @functools.partial(
        pl.kernel,
        mesh=mesh,
        out_type=jax.ShapeDtypeStruct((N,), jnp.float32),
        scratch_types=[
            pltpu.VMEM((2, D // 2), jnp.float32),
            pltpu.VMEM((_L,), jnp.int32),
            pltpu.VMEM((D,), jnp.float32),
        ]
        + [pltpu.VMEM((CH,), jnp.float32)] * NBUF
        + [pltpu.SemaphoreType.DMA] * (2 * NBUF),
    )
    def k(x_hbm, m_hbm, w_hbm, out_hbm, w_v, m_v, add_v, *rest):
        bufs = rest[:NBUF]
        lds = rest[NBUF:2 * NBUF]
        sts = rest[2 * NBUF:]
        wid = lax.axis_index("s") * _NC + lax.axis_index("c")
        base = wid * per_w
        pltpu.sync_copy(m_hbm, m_v)
        pltpu.sync_copy(w_hbm, w_v)
        m0 = m_v[pl.ds(0, _L)][0]

        half = nvec // 2
        for j in (0, 1):
            sel = j < m0

            @plsc.parallel_loop(0, half, 1, unroll=2)
            def _(jc):
                a0 = w_v[0, pl.ds(jc * _L, _L)]
                a1 = w_v[1, pl.ds(jc * _L, _L)]
                add_v[pl.ds(j * (D // 2) + jc * _L, _L)] = jnp.where(sel, a1, a0)

        for c in range(NBUF):
            pltpu.async_copy(x_hbm.at[pl.ds(base + c * CH, CH)], bufs[c], lds[c])

        def do_chunk(g, b):
            buf = bufs[b]
            pltpu.make_async_copy(x_hbm.at[pl.ds(0, CH)], buf, lds[b]).wait()

            @plsc.parallel_loop(0, nvec, 1, unroll=2)
            def _(jc):
                a = add_v[pl.ds(jc * _L, _L)]
                for r in range(RCH):
                    off = r * D + jc * _L
                    buf[pl.ds(off, _L)] = buf[pl.ds(off, _L)] + a

            pltpu.async_copy(buf, out_hbm.at[pl.ds(base + g * CH, CH)], sts[b])

            bb = (b + 2) % NBUF
            gg = g + 2

            @pl.when((gg >= NBUF) & (gg < nch))
            def _():
                pltpu.make_async_copy(
                    bufs[bb], out_hbm.at[pl.ds(0, CH)], sts[bb]
                ).wait()
                pltpu.async_copy(
                    x_hbm.at[pl.ds(base + gg * CH, CH)], bufs[bb], lds[bb]
                )

        def quad_body(g4, carry):
            for b in range(NBUF):
                do_chunk(NBUF * g4 + b, b)
            return carry

        lax.fori_loop(0, nch // NBUF, quad_body, 0)

        for b in range(NBUF):
            pltpu.make_async_copy(bufs[b], out_hbm.at[pl.ds(0, CH)], sts[b]).wait()

    return k(x1, mask16, w)


def kernel(x, mask, type_embedding_weight):
    b, n, d = x.shape
    x1 = x.reshape(b * n * d)
    mask16 = jnp.zeros((_L,), jnp.int32).at[: mask.shape[0]].set(
        mask.astype(jnp.int32)
    )
    out1 = _sc_add_kernel(x1, mask16, type_embedding_weight)
    return out1.reshape(b, n, d)


# speedup vs baseline: 3.6786x; 3.6786x over previous
import functools

import jax
import jax.numpy as jnp
from jax import lax
from jax.experimental import pallas as pl
from jax.experimental.pallas import tpu as pltpu
from jax.experimental.pallas import tpu_sc as plsc

_NC, _NS, _L = 2, 16, 16


def _sc_lookup(mask16, w):
    n_rows, d = w.shape
    mesh = plsc.VectorSubcoreMesh(core_axis_name="c", subcore_axis_name="s")

    @functools.partial(
        pl.kernel,
        mesh=mesh,
        out_type=jax.ShapeDtypeStruct((n_rows, d), jnp.float32),
        scratch_types=[
            pltpu.VMEM((_L,), jnp.int32),
            pltpu.VMEM((_L,), jnp.int32),
            pltpu.VMEM((_L, d), jnp.float32),
            pltpu.SemaphoreType.DMA,
        ],
    )
    def k(m_hbm, w_hbm, out_hbm, m_v, idx_v, rows_v, sem):
        wid = lax.axis_index("s") * _NC + lax.axis_index("c")

        @pl.when(wid == 0)
        def _():
            pltpu.sync_copy(m_hbm, m_v)
            m0 = m_v[pl.ds(0, _L)][0]
            col = lax.iota(jnp.int32, _L)
            ids = jnp.where(col < m0, 1, 0)
            idx_v[...] = ids
            pltpu.async_copy(w_hbm.at[idx_v], rows_v, sem).wait()
            pltpu.sync_copy(rows_v.at[pl.ds(0, n_rows)], out_hbm)

    return k(mask16, w)


def _tc_body(emb_ref, x_ref, o_ref):
    o_ref[...] = x_ref[...] + emb_ref[...][None, :, :]


def _tc_add(x, emb):
    b, n, d = x.shape
    tb = 1024
    return pl.pallas_call(
        _tc_body,
        grid=(b // tb,),
        in_specs=[
            pl.BlockSpec((n, d), lambda i: (0, 0)),
            pl.BlockSpec((tb, n, d), lambda i: (i, 0, 0)),
        ],
        out_specs=pl.BlockSpec((tb, n, d), lambda i: (i, 0, 0)),
        out_shape=jax.ShapeDtypeStruct((b, n, d), x.dtype),
    )(emb, x)


def kernel(x, mask, type_embedding_weight):
    mask16 = jnp.zeros((_L,), jnp.int32).at[: mask.shape[0]].set(
        mask.astype(jnp.int32)
    )
    type_emb = _sc_lookup(mask16, type_embedding_weight)
    return _tc_add(x, type_emb)


# speedup vs baseline: 3.7315x; 1.0144x over previous
import functools

import jax
import jax.numpy as jnp
from jax import lax
from jax.experimental import pallas as pl
from jax.experimental.pallas import tpu as pltpu
from jax.experimental.pallas import tpu_sc as plsc

_NC, _NS, _L = 2, 16, 16


def _sc_lookup(mask16, w):
    n_rows, d = w.shape
    mesh = plsc.VectorSubcoreMesh(
        core_axis_name="c", subcore_axis_name="s", num_cores=1
    )

    @functools.partial(
        pl.kernel,
        mesh=mesh,
        out_type=jax.ShapeDtypeStruct((n_rows, d), jnp.float32),
        scratch_types=[
            pltpu.VMEM((_L,), jnp.int32),
            pltpu.VMEM((_L,), jnp.int32),
            pltpu.VMEM((_L, d), jnp.float32),
            pltpu.SemaphoreType.DMA,
        ],
    )
    def k(m_hbm, w_hbm, out_hbm, m_v, idx_v, rows_v, sem):
        wid = lax.axis_index("s") * _NC + lax.axis_index("c")

        @pl.when(wid == 0)
        def _():
            pltpu.sync_copy(m_hbm, m_v)
            m0 = m_v[pl.ds(0, _L)][0]
            col = lax.iota(jnp.int32, _L)
            ids = jnp.where(col < m0, 1, 0)
            idx_v[...] = ids
            pltpu.async_copy(w_hbm.at[idx_v], rows_v, sem).wait()
            pltpu.sync_copy(rows_v.at[pl.ds(0, n_rows)], out_hbm)

    return k(mask16, w)


def _tc_body(emb_ref, x_ref, o_ref):
    o_ref[...] = x_ref[...] + emb_ref[...][None, :, :]


def _tc_add(x, emb):
    b, n, d = x.shape
    tb = 1024
    return pl.pallas_call(
        _tc_body,
        grid=(b // tb,),
        in_specs=[
            pl.BlockSpec((n, d), lambda i: (0, 0)),
            pl.BlockSpec((tb, n, d), lambda i: (i, 0, 0)),
        ],
        out_specs=pl.BlockSpec((tb, n, d), lambda i: (i, 0, 0)),
        out_shape=jax.ShapeDtypeStruct((b, n, d), x.dtype),
    )(emb, x)


def kernel(x, mask, type_embedding_weight):
    mask16 = jnp.zeros((_L,), jnp.int32).at[: mask.shape[0]].set(
        mask.astype(jnp.int32)
    )
    type_emb = _sc_lookup(mask16, type_embedding_weight)
    return _tc_add(x, type_emb)


# speedup vs baseline: 3.9544x; 1.0597x over previous
import functools

import jax
import jax.numpy as jnp
from jax import lax
from jax.experimental import pallas as pl
from jax.experimental.pallas import tpu as pltpu
from jax.experimental.pallas import tpu_sc as plsc

_NC, _NS, _L = 2, 16, 16
_TB = 1024
_HEAD_TILES = 4


def _sc_lookup(mask16, w):
    n_rows, d = w.shape
    mesh = plsc.VectorSubcoreMesh(
        core_axis_name="c", subcore_axis_name="s", num_cores=1
    )

    @functools.partial(
        pl.kernel,
        mesh=mesh,
        out_type=jax.ShapeDtypeStruct((n_rows, d), jnp.float32),
        scratch_types=[
            pltpu.VMEM((_L,), jnp.int32),
            pltpu.VMEM((_L,), jnp.int32),
            pltpu.VMEM((_L, d), jnp.float32),
            pltpu.SemaphoreType.DMA,
        ],
    )
    def k(m_hbm, w_hbm, out_hbm, m_v, idx_v, rows_v, sem):
        wid = lax.axis_index("s")

        @pl.when(wid == 0)
        def _():
            pltpu.sync_copy(m_hbm, m_v)
            m0 = m_v[pl.ds(0, _L)][0]
            col = lax.iota(jnp.int32, _L)
            ids = jnp.where(col < m0, 1, 0)
            idx_v[...] = ids
            pltpu.async_copy(w_hbm.at[idx_v], rows_v, sem).wait()
            pltpu.sync_copy(rows_v.at[pl.ds(0, n_rows)], out_hbm)

    return k(mask16, w)


def _tc_head_body(mask_ref, w_ref, x_ref, o_ref):
    m0 = mask_ref[0]
    n = w_ref.shape[0]
    sel = lax.broadcasted_iota(jnp.int32, (n, 1), 0) < m0
    addend = jnp.where(sel, w_ref[1:2, :], w_ref[0:1, :])
    o_ref[...] = x_ref[...] + addend[None, :, :]


def _tc_head(x, mask_i, w):
    b, n, d = x.shape
    return pl.pallas_call(
        _tc_head_body,
        grid=(_HEAD_TILES,),
        in_specs=[
            pl.BlockSpec(memory_space=pltpu.SMEM),
            pl.BlockSpec((n, d), lambda i: (0, 0)),
            pl.BlockSpec((_TB, n, d), lambda i: (i, 0, 0)),
        ],
        out_specs=pl.BlockSpec((_TB, n, d), lambda i: (i, 0, 0)),
        out_shape=jax.ShapeDtypeStruct((b, n, d), x.dtype),
    )(mask_i, w, x)


def _tc_tail_body(emb_ref, x_ref, acc_ref, o_ref):
    o_ref[...] = x_ref[...] + emb_ref[...][None, :, :]


def _tc_tail(x, emb, acc):
    b, n, d = x.shape
    ntail = b // _TB - _HEAD_TILES
    return pl.pallas_call(
        _tc_tail_body,
        grid=(ntail,),
        in_specs=[
            pl.BlockSpec((n, d), lambda i: (0, 0)),
            pl.BlockSpec((_TB, n, d), lambda i: (i + _HEAD_TILES, 0, 0)),
            pl.BlockSpec(memory_space=pl.ANY),
        ],
        out_specs=pl.BlockSpec((_TB, n, d), lambda i: (i + _HEAD_TILES, 0, 0)),
        out_shape=jax.ShapeDtypeStruct((b, n, d), x.dtype),
        input_output_aliases={2: 0},
    )(emb, x, acc)


def kernel(x, mask, type_embedding_weight):
    mask_i = mask.astype(jnp.int32)
    mask16 = jnp.zeros((_L,), jnp.int32).at[: mask.shape[0]].set(mask_i)
    type_emb = _sc_lookup(mask16, type_embedding_weight)
    acc = _tc_head(x, mask_i, type_embedding_weight)
    return _tc_tail(x, type_emb, acc)
